# Initial kernel scaffold; baseline (speedup 1.0000x reference)
import jax
import jax.numpy as jnp
from jax.experimental import pallas as pl


def kernel(x, adj, nodes, epoch, W1, b1, W2, b2):
    raise NotImplementedError("write your pallas kernel here")


# speedup vs baseline: 19.4351x; 19.4351x over previous
import functools

import jax
import jax.numpy as jnp
from jax import lax
from jax.experimental import pallas as pl
from jax.experimental.pallas import tpu as pltpu
from jax.experimental.pallas import tpu_sc as plsc

N = 10000
E = 320000
F_IN = 128
H = 128
HW = H // 2
C = 40
CP = 48
NPAD = 10240
NTILE = 32
K = 128
CHUNKS = 80
RPT = NPAD // 16

_SC_PARAMS = pltpu.CompilerParams(use_tc_tiling_on_sc=False)


@functools.cache
def _mesh():
    return plsc.VectorSubcoreMesh(core_axis_name="c", subcore_axis_name="s")


def _zero_rows(buf, nrows, ncols):
    def zb(r, carry):
        for c in range(ncols // 16):
            buf[r, pl.ds(c * 16, 16)] = jnp.zeros((16,), jnp.float32)
        return carry

    lax.fori_loop(0, nrows, zb, 0)


def _pipeline(nchunks, srcv, dstv, tab_hbm, acc, bufs, gsems, ssems):
    def g_start(j, p):
        pltpu.async_copy(tab_hbm.at[srcv.at[j]], bufs[p], gsems[p])

    def g_wait(j, p):
        pltpu.make_async_copy(tab_hbm.at[srcv.at[j]], bufs[p],
                              gsems[p]).wait()

    def s_start(j, p):
        pltpu.async_copy(bufs[p], acc.at[dstv.at[j]], ssems[p], add=True)

    def s_wait(j, p):
        pltpu.make_async_copy(bufs[p], acc.at[dstv.at[j]], ssems[p]).wait()

    g_start(0, 0)
    g_wait(0, 0); g_start(1, 1); s_start(0, 0)
    g_wait(1, 1); s_wait(0, 0); g_start(2, 0); s_start(1, 1)

    def step(g, carry):
        j = 2 * g
        g_wait(j, 0); s_wait(j - 1, 1); g_start(j + 1, 1); s_start(j, 0)
        g_wait(j + 1, 1); s_wait(j, 0); g_start(j + 2, 0); s_start(j + 1, 1)
        return carry

    lax.fori_loop(1, nchunks // 2 - 1, step, 0)
    j = nchunks - 2
    g_wait(j, 0); s_wait(j - 1, 1); g_start(j + 1, 1); s_start(j, 0)
    g_wait(j + 1, 1); s_wait(j, 0); s_start(j + 1, 1)
    s_wait(j + 1, 1)


def _degrees_call(src_t, dst_t):
    def body(src_hbm, dst_hbm, degi_hbm, dego_hbm, srcv, dstv, onesv, zbuf,
             acc_i, acc_o):
        cid = lax.axis_index("c")
        sid = lax.axis_index("s")
        wid = cid * 16 + sid
        pltpu.sync_copy(src_hbm.at[wid], srcv)
        pltpu.sync_copy(dst_hbm.at[wid], dstv)
        for i in range(K // 16):
            onesv[pl.ds(i * 16, 16)] = jnp.full((16,), 1.0, jnp.float32)

        def zb(i, carry):
            zbuf[pl.ds(i * 16, 16)] = jnp.zeros((16,), jnp.float32)
            return carry

        lax.fori_loop(0, RPT // 16, zb, 0)
        pltpu.sync_copy(zbuf, acc_i.at[pl.ds(sid * RPT, RPT)])
        pltpu.sync_copy(zbuf, acc_o.at[pl.ds(sid * RPT, RPT)])
        plsc.subcore_barrier()

        def chunk(j, carry):
            pltpu.sync_copy(onesv, acc_i.at[dstv.at[j]], add=True)
            pltpu.sync_copy(onesv, acc_o.at[srcv.at[j]], add=True)
            return carry

        lax.fori_loop(0, CHUNKS, chunk, 0)
        plsc.subcore_barrier()
        sl = pl.ds(sid * RPT, RPT)
        pltpu.sync_copy(acc_i.at[sl], zbuf)
        pltpu.sync_copy(zbuf, degi_hbm.at[cid, sl])
        pltpu.sync_copy(acc_o.at[sl], zbuf)
        pltpu.sync_copy(zbuf, dego_hbm.at[cid, sl])

    f = pl.kernel(
        body,
        out_type=(jax.ShapeDtypeStruct((2, NPAD), jnp.float32),
                  jax.ShapeDtypeStruct((2, NPAD), jnp.float32)),
        mesh=_mesh(),
        scratch_types=[
            pltpu.VMEM((CHUNKS, K), jnp.int32),
            pltpu.VMEM((CHUNKS, K), jnp.int32),
            pltpu.VMEM((K,), jnp.float32),
            pltpu.VMEM((RPT,), jnp.float32),
            pltpu.VMEM_SHARED((NPAD,), jnp.float32),
            pltpu.VMEM_SHARED((NPAD,), jnp.float32),
        ],
    )
    return f(src_t, dst_t)


def _msgpass_feature_split(tab2, src_t2, dst_t2):
    nch = 2 * CHUNKS

    def body(tab_hbm, src_hbm, dst_hbm, out_hbm, srcv, dstv, b0, b1, acc,
             sg0, sg1, ss0, ss1):
        cid = lax.axis_index("c")
        sid = lax.axis_index("s")
        pltpu.sync_copy(src_hbm.at[sid], srcv)
        pltpu.sync_copy(dst_hbm.at[sid], dstv)
        off = cid * NPAD

        def adj(r, carry):
            for c in range(K // 16):
                s = pl.ds(c * 16, 16)
                srcv[r, s] = srcv[r, s] + off
            return carry

        lax.fori_loop(0, nch, adj, 0)
        _zero_rows(b0, K, HW)
        for t in range(RPT // K):
            pltpu.sync_copy(b0, acc.at[pl.ds(sid * RPT + t * K, K)])
        plsc.subcore_barrier()
        _pipeline(nch, srcv, dstv, tab_hbm, acc, (b0, b1), (sg0, sg1),
                  (ss0, ss1))
        plsc.subcore_barrier()
        for t in range(RPT // K):
            sl = pl.ds(sid * RPT + t * K, K)
            pltpu.sync_copy(acc.at[sl], b0)
            pltpu.sync_copy(b0, out_hbm.at[cid, sl])

    f = pl.kernel(
        body,
        out_type=jax.ShapeDtypeStruct((2, NPAD, HW), jnp.float32),
        mesh=_mesh(),
        scratch_types=[
            pltpu.VMEM((nch, K), jnp.int32),
            pltpu.VMEM((nch, K), jnp.int32),
            pltpu.VMEM((K, HW), jnp.float32),
            pltpu.VMEM((K, HW), jnp.float32),
            pltpu.VMEM_SHARED((NPAD, HW), jnp.float32),
            pltpu.SemaphoreType.DMA,
            pltpu.SemaphoreType.DMA,
            pltpu.SemaphoreType.DMA,
            pltpu.SemaphoreType.DMA,
        ],
        compiler_params=_SC_PARAMS,
    )
    return f(tab2, src_t2, dst_t2)


def _msgpass_edge_split(tab, src_t, dst_t):
    def body(tab_hbm, src_hbm, dst_hbm, out_hbm, srcv, dstv, b0, b1, acc,
             sg0, sg1, ss0, ss1):
        cid = lax.axis_index("c")
        sid = lax.axis_index("s")
        wid = cid * 16 + sid
        pltpu.sync_copy(src_hbm.at[wid], srcv)
        pltpu.sync_copy(dst_hbm.at[wid], dstv)
        _zero_rows(b0, K, CP)
        for t in range(RPT // K):
            pltpu.sync_copy(b0, acc.at[pl.ds(sid * RPT + t * K, K)])
        plsc.subcore_barrier()
        _pipeline(CHUNKS, srcv, dstv, tab_hbm, acc, (b0, b1), (sg0, sg1),
                  (ss0, ss1))
        plsc.subcore_barrier()
        for t in range(RPT // K):
            sl = pl.ds(sid * RPT + t * K, K)
            pltpu.sync_copy(acc.at[sl], b0)
            pltpu.sync_copy(b0, out_hbm.at[cid, sl])

    f = pl.kernel(
        body,
        out_type=jax.ShapeDtypeStruct((2, NPAD, CP), jnp.float32),
        mesh=_mesh(),
        scratch_types=[
            pltpu.VMEM((CHUNKS, K), jnp.int32),
            pltpu.VMEM((CHUNKS, K), jnp.int32),
            pltpu.VMEM((K, CP), jnp.float32),
            pltpu.VMEM((K, CP), jnp.float32),
            pltpu.VMEM_SHARED((NPAD, CP), jnp.float32),
            pltpu.SemaphoreType.DMA,
            pltpu.SemaphoreType.DMA,
            pltpu.SemaphoreType.DMA,
            pltpu.SemaphoreType.DMA,
        ],
        compiler_params=_SC_PARAMS,
    )
    return f(tab, src_t, dst_t)


def _mm_scale(x_pad, W1, dego0, dego1):
    def body(x_ref, w_ref, d0, d1, o_ref):
        dinv = lax.rsqrt(d0[...] + d1[...] + 1.0)
        h = jnp.dot(x_ref[...], w_ref[...],
                    preferred_element_type=jnp.float32) * dinv
        o_ref[0, :, :] = h[:, :HW]
        o_ref[1, :, :] = h[:, HW:]

    return pl.pallas_call(
        body,
        grid=(NPAD // 1024,),
        in_specs=[
            pl.BlockSpec((1024, F_IN), lambda i: (i, 0)),
            pl.BlockSpec((F_IN, H), lambda i: (0, 0)),
            pl.BlockSpec((1024, 1), lambda i: (i, 0)),
            pl.BlockSpec((1024, 1), lambda i: (i, 0)),
        ],
        out_specs=pl.BlockSpec((2, 1024, HW), lambda i: (0, i, 0)),
        out_shape=jax.ShapeDtypeStruct((2, NPAD, HW), jnp.float32),
    )(x_pad, W1, dego0, dego1)


def _mid(p_lo, p_hi, degi0, degi1, dego0, dego1, b1, W2p):
    def body(pl_ref, ph_ref, di0, di1, do0, do1, b_ref, w_ref, o_ref):
        dinv_i = lax.rsqrt(di0[...] + di1[...] + 1.0)
        agg = jnp.concatenate([pl_ref[...], ph_ref[...]], axis=1)
        a = agg * dinv_i + b_ref[...]
        h = jnp.maximum(a, 0.0)
        dinv_o = lax.rsqrt(do0[...] + do1[...] + 1.0)
        o_ref[...] = jnp.dot(h, w_ref[...],
                             preferred_element_type=jnp.float32) * dinv_o

    return pl.pallas_call(
        body,
        grid=(NPAD // 1024,),
        in_specs=[
            pl.BlockSpec((1024, HW), lambda i: (i, 0)),
            pl.BlockSpec((1024, HW), lambda i: (i, 0)),
            pl.BlockSpec((1024, 1), lambda i: (i, 0)),
            pl.BlockSpec((1024, 1), lambda i: (i, 0)),
            pl.BlockSpec((1024, 1), lambda i: (i, 0)),
            pl.BlockSpec((1024, 1), lambda i: (i, 0)),
            pl.BlockSpec((1, H), lambda i: (0, 0)),
            pl.BlockSpec((H, CP), lambda i: (0, 0)),
        ],
        out_specs=pl.BlockSpec((1024, CP), lambda i: (i, 0)),
        out_shape=jax.ShapeDtypeStruct((NPAD, CP), jnp.float32),
    )(p_lo, p_hi, degi0, degi1, dego0, dego1, b1, W2p)


def _final(q0, q1, degi0, degi1, b2):
    def body(q0_ref, q1_ref, di0, di1, b_ref, o_ref):
        dinv_i = lax.rsqrt(di0[...] + di1[...] + 1.0)
        z = (q0_ref[...] + q1_ref[...])[:, :C] * dinv_i + b_ref[...]
        m = jnp.max(z, axis=1, keepdims=True)
        e = jnp.exp(z - m)
        s = jnp.sum(e, axis=1, keepdims=True)
        o_ref[...] = z - m - jnp.log(s)

    return pl.pallas_call(
        body,
        grid=(10,),
        in_specs=[
            pl.BlockSpec((1000, CP), lambda i: (i, 0)),
            pl.BlockSpec((1000, CP), lambda i: (i, 0)),
            pl.BlockSpec((1000, 1), lambda i: (i, 0)),
            pl.BlockSpec((1000, 1), lambda i: (i, 0)),
            pl.BlockSpec((1, C), lambda i: (0, 0)),
        ],
        out_specs=pl.BlockSpec((1000, C), lambda i: (i, 0)),
        out_shape=jax.ShapeDtypeStruct((N, C), jnp.float32),
    )(q0, q1, degi0, degi1, b2)


def kernel(x, adj, nodes, epoch, W1, b1, W2, b2):
    src = adj[0]
    dst = adj[1]
    ept = E // NTILE
    npad_e = CHUNKS * K - ept
    pad = (jnp.arange(npad_e, dtype=jnp.int32) + N)[None, :]
    pad = jnp.broadcast_to(pad, (NTILE, npad_e))
    src_t = jnp.concatenate([src.reshape(NTILE, ept), pad], axis=1)
    src_t = src_t.reshape(NTILE, CHUNKS, K)
    dst_t = jnp.concatenate([dst.reshape(NTILE, ept), pad], axis=1)
    dst_t = dst_t.reshape(NTILE, CHUNKS, K)
    src_t2 = src_t.reshape(16, 2 * CHUNKS, K)
    dst_t2 = dst_t.reshape(16, 2 * CHUNKS, K)

    x_pad = jnp.pad(x, ((0, NPAD - N), (0, 0)))
    W2p = jnp.pad(W2, ((0, 0), (0, CP - C)))

    degi_p, dego_p = _degrees_call(src_t, dst_t)
    degi0 = degi_p[0].reshape(NPAD, 1)
    degi1 = degi_p[1].reshape(NPAD, 1)
    dego0 = dego_p[0].reshape(NPAD, 1)
    dego1 = dego_p[1].reshape(NPAD, 1)

    h1s = _mm_scale(x_pad, W1, dego0, dego1).reshape(2 * NPAD, HW)
    p = _msgpass_feature_split(h1s, src_t2, dst_t2)
    h2s = _mid(p[0], p[1], degi0, degi1, dego0, dego1, b1, W2p)
    q = _msgpass_edge_split(h2s, src_t, dst_t)
    return _final(q[0], q[1], degi0, degi1, b2)


# speedup vs baseline: 20.7504x; 1.0677x over previous
import functools

import jax
import jax.numpy as jnp
from jax import lax
from jax.experimental import pallas as pl
from jax.experimental.pallas import tpu as pltpu
from jax.experimental.pallas import tpu_sc as plsc

N = 10000
E = 320000
F_IN = 128
H = 128
HW = H // 2
C = 40
CP = 48
NPAD = 10240
NTILE = 32
K = 128
CHUNKS = 80
KM = 96
CM = 108
RPT = NPAD // 16

_SC_PARAMS = pltpu.CompilerParams(use_tc_tiling_on_sc=False)


@functools.cache
def _mesh():
    return plsc.VectorSubcoreMesh(core_axis_name="c", subcore_axis_name="s")


def _zero_rows(buf, nrows, ncols):
    def zb(r, carry):
        for c in range(ncols // 16):
            buf[r, pl.ds(c * 16, 16)] = jnp.zeros((16,), jnp.float32)
        return carry

    lax.fori_loop(0, nrows, zb, 0)


def _pipeline(nchunks, srcv, dstv, tab_hbm, acc, bufs, gsems, ssems):
    def g_start(j, p):
        pltpu.async_copy(tab_hbm.at[srcv.at[j]], bufs[p], gsems[p])

    def g_wait(j, p):
        pltpu.make_async_copy(tab_hbm.at[srcv.at[j]], bufs[p],
                              gsems[p]).wait()

    def s_start(j, p):
        pltpu.async_copy(bufs[p], acc.at[dstv.at[j]], ssems[p], add=True)

    def s_wait(j, p):
        pltpu.make_async_copy(bufs[p], acc.at[dstv.at[j]], ssems[p]).wait()

    g_start(0, 0)
    g_wait(0, 0); g_start(1, 1); s_start(0, 0)
    g_wait(1, 1); s_wait(0, 0); g_start(2, 0); s_start(1, 1)

    def step(g, carry):
        j = 2 * g
        g_wait(j, 0); s_wait(j - 1, 1); g_start(j + 1, 1); s_start(j, 0)
        g_wait(j + 1, 1); s_wait(j, 0); g_start(j + 2, 0); s_start(j + 1, 1)
        return carry

    lax.fori_loop(1, nchunks // 2 - 1, step, 0)
    j = nchunks - 2
    g_wait(j, 0); s_wait(j - 1, 1); g_start(j + 1, 1); s_start(j, 0)
    g_wait(j + 1, 1); s_wait(j, 0); s_start(j + 1, 1)
    s_wait(j + 1, 1)


def _degrees_call(src_t, dst_t):
    def body(src_hbm, dst_hbm, degi_hbm, dego_hbm, srcv, dstv, onesv, zbuf,
             acc_i, acc_o):
        cid = lax.axis_index("c")
        sid = lax.axis_index("s")
        wid = cid * 16 + sid
        pltpu.sync_copy(src_hbm.at[wid], srcv)
        pltpu.sync_copy(dst_hbm.at[wid], dstv)
        for i in range(K // 16):
            onesv[pl.ds(i * 16, 16)] = jnp.full((16,), 1.0, jnp.float32)

        def zb(i, carry):
            zbuf[pl.ds(i * 16, 16)] = jnp.zeros((16,), jnp.float32)
            return carry

        lax.fori_loop(0, RPT // 16, zb, 0)
        pltpu.sync_copy(zbuf, acc_i.at[pl.ds(sid * RPT, RPT)])
        pltpu.sync_copy(zbuf, acc_o.at[pl.ds(sid * RPT, RPT)])
        plsc.subcore_barrier()

        def chunk(j, carry):
            pltpu.sync_copy(onesv, acc_i.at[dstv.at[j]], add=True)
            pltpu.sync_copy(onesv, acc_o.at[srcv.at[j]], add=True)
            return carry

        lax.fori_loop(0, CHUNKS, chunk, 0)
        plsc.subcore_barrier()
        sl = pl.ds(sid * RPT, RPT)
        pltpu.sync_copy(acc_i.at[sl], zbuf)
        pltpu.sync_copy(zbuf, degi_hbm.at[cid, sl])
        pltpu.sync_copy(acc_o.at[sl], zbuf)
        pltpu.sync_copy(zbuf, dego_hbm.at[cid, sl])

    f = pl.kernel(
        body,
        out_type=(jax.ShapeDtypeStruct((2, NPAD), jnp.float32),
                  jax.ShapeDtypeStruct((2, NPAD), jnp.float32)),
        mesh=_mesh(),
        scratch_types=[
            pltpu.VMEM((CHUNKS, K), jnp.int32),
            pltpu.VMEM((CHUNKS, K), jnp.int32),
            pltpu.VMEM((K,), jnp.float32),
            pltpu.VMEM((RPT,), jnp.float32),
            pltpu.VMEM_SHARED((NPAD,), jnp.float32),
            pltpu.VMEM_SHARED((NPAD,), jnp.float32),
        ],
    )
    return f(src_t, dst_t)


def _msgpass_edge_split(tab, src_t, dst_t, width):
    ex_chunks = [(t * KM, KM) for t in range(RPT // KM)]
    if RPT % KM:
        ex_chunks.append((RPT - RPT % KM, RPT % KM))

    def body(tab_hbm, src_hbm, dst_hbm, out_hbm, srcv, dstv, b0, b1, acc,
             sg0, sg1, ss0, ss1):
        cid = lax.axis_index("c")
        sid = lax.axis_index("s")
        wid = cid * 16 + sid
        pltpu.sync_copy(src_hbm.at[wid], srcv)
        pltpu.sync_copy(dst_hbm.at[wid], dstv)
        _zero_rows(b0, KM, width)
        for t0, n in ex_chunks:
            pltpu.sync_copy(b0.at[pl.ds(0, n)],
                            acc.at[pl.ds(sid * RPT + t0, n)])
        plsc.subcore_barrier()
        _pipeline(CM, srcv, dstv, tab_hbm, acc, (b0, b1), (sg0, sg1),
                  (ss0, ss1))
        plsc.subcore_barrier()
        for t0, n in ex_chunks:
            sl = pl.ds(sid * RPT + t0, n)
            pltpu.sync_copy(acc.at[sl], b0.at[pl.ds(0, n)])
            pltpu.sync_copy(b0.at[pl.ds(0, n)], out_hbm.at[cid, sl])

    f = pl.kernel(
        body,
        out_type=jax.ShapeDtypeStruct((2, NPAD, width), jnp.float32),
        mesh=_mesh(),
        scratch_types=[
            pltpu.VMEM((CM, KM), jnp.int32),
            pltpu.VMEM((CM, KM), jnp.int32),
            pltpu.VMEM((KM, width), jnp.float32),
            pltpu.VMEM((KM, width), jnp.float32),
            pltpu.VMEM_SHARED((NPAD, width), jnp.float32),
            pltpu.SemaphoreType.DMA,
            pltpu.SemaphoreType.DMA,
            pltpu.SemaphoreType.DMA,
            pltpu.SemaphoreType.DMA,
        ],
        compiler_params=_SC_PARAMS,
    )
    return f(tab, src_t, dst_t)


def _mm_scale(x_pad, W1, dego0, dego1):
    def body(x_ref, w_ref, d0, d1, o_ref):
        dinv = lax.rsqrt(d0[...] + d1[...] + 1.0)
        o_ref[...] = jnp.dot(x_ref[...], w_ref[...],
                             preferred_element_type=jnp.float32) * dinv

    return pl.pallas_call(
        body,
        grid=(NPAD // 1024,),
        in_specs=[
            pl.BlockSpec((1024, F_IN), lambda i: (i, 0)),
            pl.BlockSpec((F_IN, H), lambda i: (0, 0)),
            pl.BlockSpec((1024, 1), lambda i: (i, 0)),
            pl.BlockSpec((1024, 1), lambda i: (i, 0)),
        ],
        out_specs=pl.BlockSpec((1024, H), lambda i: (i, 0)),
        out_shape=jax.ShapeDtypeStruct((NPAD, H), jnp.float32),
    )(x_pad, W1, dego0, dego1)


def _mid(p0, p1, degi0, degi1, dego0, dego1, b1, W2p):
    def body(p0_ref, p1_ref, di0, di1, do0, do1, b_ref, w_ref, o_ref):
        dinv_i = lax.rsqrt(di0[...] + di1[...] + 1.0)
        a = (p0_ref[...] + p1_ref[...]) * dinv_i + b_ref[...]
        h = jnp.maximum(a, 0.0)
        dinv_o = lax.rsqrt(do0[...] + do1[...] + 1.0)
        o_ref[...] = jnp.dot(h, w_ref[...],
                             preferred_element_type=jnp.float32) * dinv_o

    return pl.pallas_call(
        body,
        grid=(NPAD // 1024,),
        in_specs=[
            pl.BlockSpec((1024, H), lambda i: (i, 0)),
            pl.BlockSpec((1024, H), lambda i: (i, 0)),
            pl.BlockSpec((1024, 1), lambda i: (i, 0)),
            pl.BlockSpec((1024, 1), lambda i: (i, 0)),
            pl.BlockSpec((1024, 1), lambda i: (i, 0)),
            pl.BlockSpec((1024, 1), lambda i: (i, 0)),
            pl.BlockSpec((1, H), lambda i: (0, 0)),
            pl.BlockSpec((H, CP), lambda i: (0, 0)),
        ],
        out_specs=pl.BlockSpec((1024, CP), lambda i: (i, 0)),
        out_shape=jax.ShapeDtypeStruct((NPAD, CP), jnp.float32),
    )(p0, p1, degi0, degi1, dego0, dego1, b1, W2p)


def _final(q0, q1, degi0, degi1, b2):
    def body(q0_ref, q1_ref, di0, di1, b_ref, o_ref):
        dinv_i = lax.rsqrt(di0[...] + di1[...] + 1.0)
        z = (q0_ref[...] + q1_ref[...])[:, :C] * dinv_i + b_ref[...]
        m = jnp.max(z, axis=1, keepdims=True)
        e = jnp.exp(z - m)
        s = jnp.sum(e, axis=1, keepdims=True)
        o_ref[...] = z - m - jnp.log(s)

    return pl.pallas_call(
        body,
        grid=(10,),
        in_specs=[
            pl.BlockSpec((1000, CP), lambda i: (i, 0)),
            pl.BlockSpec((1000, CP), lambda i: (i, 0)),
            pl.BlockSpec((1000, 1), lambda i: (i, 0)),
            pl.BlockSpec((1000, 1), lambda i: (i, 0)),
            pl.BlockSpec((1, C), lambda i: (0, 0)),
        ],
        out_specs=pl.BlockSpec((1000, C), lambda i: (i, 0)),
        out_shape=jax.ShapeDtypeStruct((N, C), jnp.float32),
    )(q0, q1, degi0, degi1, b2)


def kernel(x, adj, nodes, epoch, W1, b1, W2, b2):
    src = adj[0]
    dst = adj[1]
    ept = E // NTILE

    def shard(idx, chunks, k):
        npad_e = chunks * k - ept
        pad = (jnp.arange(npad_e, dtype=jnp.int32) % (NPAD - N) + N)[None, :]
        pad = jnp.broadcast_to(pad, (NTILE, npad_e))
        t = jnp.concatenate([idx.reshape(NTILE, ept), pad], axis=1)
        return t.reshape(NTILE, chunks, k)

    src_t = shard(src, CHUNKS, K)
    dst_t = shard(dst, CHUNKS, K)
    src_m = shard(src, CM, KM)
    dst_m = shard(dst, CM, KM)

    x_pad = jnp.pad(x, ((0, NPAD - N), (0, 0)))
    W2p = jnp.pad(W2, ((0, 0), (0, CP - C)))

    degi_p, dego_p = _degrees_call(src_t, dst_t)
    degi0 = degi_p[0].reshape(NPAD, 1)
    degi1 = degi_p[1].reshape(NPAD, 1)
    dego0 = dego_p[0].reshape(NPAD, 1)
    dego1 = dego_p[1].reshape(NPAD, 1)

    h1s = _mm_scale(x_pad, W1, dego0, dego1)
    p = _msgpass_edge_split(h1s, src_m, dst_m, H)
    h2s = _mid(p[0], p[1], degi0, degi1, dego0, dego1, b1, W2p)
    q = _msgpass_edge_split(h2s, src_m, dst_m, CP)
    return _final(q[0], q[1], degi0, degi1, b2)


# speedup vs baseline: 25.9790x; 1.2520x over previous
import functools

import jax
import jax.numpy as jnp
from jax import lax
from jax.experimental import pallas as pl
from jax.experimental.pallas import tpu as pltpu
from jax.experimental.pallas import tpu_sc as plsc

N = 10000
E = 320000
F_IN = 128
H = 128
HW = H // 2
C = 40
CP = 48
NPAD = 10240
NTILE = 32
K = 128
CHUNKS = 80
KM = 48
CM = 216
NBUF = 4
RPT = NPAD // 16

_SC_PARAMS = pltpu.CompilerParams(use_tc_tiling_on_sc=False)


@functools.cache
def _mesh():
    return plsc.VectorSubcoreMesh(core_axis_name="c", subcore_axis_name="s")


def _zero_rows(buf, nrows, ncols):
    def zb(r, carry):
        for c in range(ncols // 16):
            buf[r, pl.ds(c * 16, 16)] = jnp.zeros((16,), jnp.float32)
        return carry

    lax.fori_loop(0, nrows, zb, 0)


def _pipeline(nchunks, srcv, dstv, tab_hbm, acc, bufs, gsems, ssems):
    assert nchunks % NBUF == 0 and nchunks >= 3 * NBUF

    def g_start(j, p):
        pltpu.async_copy(tab_hbm.at[srcv.at[j]], bufs[p], gsems[p])

    def g_wait(j, p):
        pltpu.make_async_copy(tab_hbm.at[srcv.at[j]], bufs[p],
                              gsems[p]).wait()

    def s_start(j, p):
        pltpu.async_copy(bufs[p], acc.at[dstv.at[j]], ssems[p], add=True)

    def s_wait(j, p):
        pltpu.make_async_copy(bufs[p], acc.at[dstv.at[j]], ssems[p]).wait()

    LA = NBUF - 1

    def iter_full(j, p):
        g_wait(j, p)
        s_wait(j - 1, (p - 1) % NBUF)
        s_start(j, p)
        g_start(j + LA, (p + LA) % NBUF)

    for p in range(LA):
        g_start(p, p)
    g_wait(0, 0); s_start(0, 0); g_start(LA, LA)
    for j in range(1, NBUF):
        iter_full(j, j % NBUF)

    def step(g, carry):
        j0 = NBUF * g
        for r in range(NBUF):
            iter_full(j0 + r, r)
        return carry

    lax.fori_loop(1, nchunks // NBUF - 2, step, 0)
    for j in range(nchunks - 2 * NBUF, nchunks):
        p = j % NBUF
        if j + LA < nchunks:
            iter_full(j, p)
        else:
            g_wait(j, p); s_wait(j - 1, (p - 1) % NBUF); s_start(j, p)
    s_wait(nchunks - 1, (nchunks - 1) % NBUF)


def _degrees_call(src_t, dst_t):
    def body(src_hbm, dst_hbm, degi_hbm, dego_hbm, srcv, dstv, onesv, zbuf,
             acc_i, acc_o):
        cid = lax.axis_index("c")
        sid = lax.axis_index("s")
        wid = cid * 16 + sid
        pltpu.sync_copy(src_hbm.at[wid], srcv)
        pltpu.sync_copy(dst_hbm.at[wid], dstv)
        for i in range(K // 16):
            onesv[pl.ds(i * 16, 16)] = jnp.full((16,), 1.0, jnp.float32)

        def zb(i, carry):
            zbuf[pl.ds(i * 16, 16)] = jnp.zeros((16,), jnp.float32)
            return carry

        lax.fori_loop(0, RPT // 16, zb, 0)
        pltpu.sync_copy(zbuf, acc_i.at[pl.ds(sid * RPT, RPT)])
        pltpu.sync_copy(zbuf, acc_o.at[pl.ds(sid * RPT, RPT)])
        plsc.subcore_barrier()

        def chunk(j, carry):
            pltpu.sync_copy(onesv, acc_i.at[dstv.at[j]], add=True)
            pltpu.sync_copy(onesv, acc_o.at[srcv.at[j]], add=True)
            return carry

        lax.fori_loop(0, CHUNKS, chunk, 0)
        plsc.subcore_barrier()
        sl = pl.ds(sid * RPT, RPT)
        pltpu.sync_copy(acc_i.at[sl], zbuf)
        pltpu.sync_copy(zbuf, degi_hbm.at[cid, sl])
        pltpu.sync_copy(acc_o.at[sl], zbuf)
        pltpu.sync_copy(zbuf, dego_hbm.at[cid, sl])

    f = pl.kernel(
        body,
        out_type=(jax.ShapeDtypeStruct((2, NPAD), jnp.float32),
                  jax.ShapeDtypeStruct((2, NPAD), jnp.float32)),
        mesh=_mesh(),
        scratch_types=[
            pltpu.VMEM((CHUNKS, K), jnp.int32),
            pltpu.VMEM((CHUNKS, K), jnp.int32),
            pltpu.VMEM((K,), jnp.float32),
            pltpu.VMEM((RPT,), jnp.float32),
            pltpu.VMEM_SHARED((NPAD,), jnp.float32),
            pltpu.VMEM_SHARED((NPAD,), jnp.float32),
        ],
    )
    return f(src_t, dst_t)


def _msgpass_edge_split(tab, src_t, dst_t, width):
    ex_chunks = [(t * KM, KM) for t in range(RPT // KM)]
    if RPT % KM:
        ex_chunks.append((RPT - RPT % KM, RPT % KM))

    def body(tab_hbm, src_hbm, dst_hbm, out_hbm, srcv, dstv, *rest):
        bufs = rest[:NBUF]
        acc = rest[NBUF]
        gsems = rest[NBUF + 1:2 * NBUF + 1]
        ssems = rest[2 * NBUF + 1:]
        b0 = bufs[0]
        cid = lax.axis_index("c")
        sid = lax.axis_index("s")
        wid = cid * 16 + sid
        pltpu.sync_copy(src_hbm.at[wid], srcv)
        pltpu.sync_copy(dst_hbm.at[wid], dstv)
        _zero_rows(b0, KM, width)
        for t0, n in ex_chunks:
            pltpu.sync_copy(b0.at[pl.ds(0, n)],
                            acc.at[pl.ds(sid * RPT + t0, n)])
        plsc.subcore_barrier()
        _pipeline(CM, srcv, dstv, tab_hbm, acc, bufs, gsems, ssems)
        plsc.subcore_barrier()
        for t0, n in ex_chunks:
            sl = pl.ds(sid * RPT + t0, n)
            pltpu.sync_copy(acc.at[sl], b0.at[pl.ds(0, n)])
            pltpu.sync_copy(b0.at[pl.ds(0, n)], out_hbm.at[cid, sl])

    f = pl.kernel(
        body,
        out_type=jax.ShapeDtypeStruct((2, NPAD, width), jnp.float32),
        mesh=_mesh(),
        scratch_types=(
            [pltpu.VMEM((CM, KM), jnp.int32),
             pltpu.VMEM((CM, KM), jnp.int32)]
            + [pltpu.VMEM((KM, width), jnp.float32)] * NBUF
            + [pltpu.VMEM_SHARED((NPAD, width), jnp.float32)]
            + [pltpu.SemaphoreType.DMA] * (2 * NBUF)
        ),
        compiler_params=_SC_PARAMS,
    )
    return f(tab, src_t, dst_t)


def _mm_scale(x_pad, W1, dego0, dego1):
    def body(x_ref, w_ref, d0, d1, o_ref):
        dinv = lax.rsqrt(d0[...] + d1[...] + 1.0)
        o_ref[...] = jnp.dot(x_ref[...], w_ref[...],
                             preferred_element_type=jnp.float32) * dinv

    return pl.pallas_call(
        body,
        grid=(NPAD // 1024,),
        in_specs=[
            pl.BlockSpec((1024, F_IN), lambda i: (i, 0)),
            pl.BlockSpec((F_IN, H), lambda i: (0, 0)),
            pl.BlockSpec((1024, 1), lambda i: (i, 0)),
            pl.BlockSpec((1024, 1), lambda i: (i, 0)),
        ],
        out_specs=pl.BlockSpec((1024, H), lambda i: (i, 0)),
        out_shape=jax.ShapeDtypeStruct((NPAD, H), jnp.float32),
    )(x_pad, W1, dego0, dego1)


def _mid(p0, p1, degi0, degi1, dego0, dego1, b1, W2p):
    def body(p0_ref, p1_ref, di0, di1, do0, do1, b_ref, w_ref, o_ref):
        dinv_i = lax.rsqrt(di0[...] + di1[...] + 1.0)
        a = (p0_ref[...] + p1_ref[...]) * dinv_i + b_ref[...]
        h = jnp.maximum(a, 0.0)
        dinv_o = lax.rsqrt(do0[...] + do1[...] + 1.0)
        o_ref[...] = jnp.dot(h, w_ref[...],
                             preferred_element_type=jnp.float32) * dinv_o

    return pl.pallas_call(
        body,
        grid=(NPAD // 1024,),
        in_specs=[
            pl.BlockSpec((1024, H), lambda i: (i, 0)),
            pl.BlockSpec((1024, H), lambda i: (i, 0)),
            pl.BlockSpec((1024, 1), lambda i: (i, 0)),
            pl.BlockSpec((1024, 1), lambda i: (i, 0)),
            pl.BlockSpec((1024, 1), lambda i: (i, 0)),
            pl.BlockSpec((1024, 1), lambda i: (i, 0)),
            pl.BlockSpec((1, H), lambda i: (0, 0)),
            pl.BlockSpec((H, CP), lambda i: (0, 0)),
        ],
        out_specs=pl.BlockSpec((1024, CP), lambda i: (i, 0)),
        out_shape=jax.ShapeDtypeStruct((NPAD, CP), jnp.float32),
    )(p0, p1, degi0, degi1, dego0, dego1, b1, W2p)


def _final(q0, q1, degi0, degi1, b2):
    def body(q0_ref, q1_ref, di0, di1, b_ref, o_ref):
        dinv_i = lax.rsqrt(di0[...] + di1[...] + 1.0)
        z = (q0_ref[...] + q1_ref[...])[:, :C] * dinv_i + b_ref[...]
        m = jnp.max(z, axis=1, keepdims=True)
        e = jnp.exp(z - m)
        s = jnp.sum(e, axis=1, keepdims=True)
        o_ref[...] = z - m - jnp.log(s)

    return pl.pallas_call(
        body,
        grid=(10,),
        in_specs=[
            pl.BlockSpec((1000, CP), lambda i: (i, 0)),
            pl.BlockSpec((1000, CP), lambda i: (i, 0)),
            pl.BlockSpec((1000, 1), lambda i: (i, 0)),
            pl.BlockSpec((1000, 1), lambda i: (i, 0)),
            pl.BlockSpec((1, C), lambda i: (0, 0)),
        ],
        out_specs=pl.BlockSpec((1000, C), lambda i: (i, 0)),
        out_shape=jax.ShapeDtypeStruct((N, C), jnp.float32),
    )(q0, q1, degi0, degi1, b2)


def kernel(x, adj, nodes, epoch, W1, b1, W2, b2):
    src = adj[0]
    dst = adj[1]
    ept = E // NTILE

    def shard(idx, chunks, k):
        npad_e = chunks * k - ept
        pad = (jnp.arange(npad_e, dtype=jnp.int32) % (NPAD - N) + N)[None, :]
        pad = jnp.broadcast_to(pad, (NTILE, npad_e))
        t = jnp.concatenate([idx.reshape(NTILE, ept), pad], axis=1)
        return t.reshape(NTILE, chunks, k)

    src_t = shard(src, CHUNKS, K)
    dst_t = shard(dst, CHUNKS, K)
    src_m = shard(src, CM, KM)
    dst_m = shard(dst, CM, KM)

    x_pad = jnp.pad(x, ((0, NPAD - N), (0, 0)))
    W2p = jnp.pad(W2, ((0, 0), (0, CP - C)))

    degi_p, dego_p = _degrees_call(src_t, dst_t)
    degi0 = degi_p[0].reshape(NPAD, 1)
    degi1 = degi_p[1].reshape(NPAD, 1)
    dego0 = dego_p[0].reshape(NPAD, 1)
    dego1 = dego_p[1].reshape(NPAD, 1)

    h1s = _mm_scale(x_pad, W1, dego0, dego1)
    p = _msgpass_edge_split(h1s, src_m, dst_m, H)
    h2s = _mid(p[0], p[1], degi0, degi1, dego0, dego1, b1, W2p)
    q = _msgpass_edge_split(h2s, src_m, dst_m, CP)
    return _final(q[0], q[1], degi0, degi1, b2)


# speedup vs baseline: 27.3495x; 1.0528x over previous
import functools

import jax
import jax.numpy as jnp
from jax import lax
from jax.experimental import pallas as pl
from jax.experimental.pallas import tpu as pltpu
from jax.experimental.pallas import tpu_sc as plsc

N = 10000
E = 320000
F_IN = 128
H = 128
HW = H // 2
C = 40
CP = 48
NPAD = 10240
NTILE = 32
K = 128
CHUNKS = 80
KM = 32
CM = 324
NBUF = 6
RPT = NPAD // 16

_SC_PARAMS = pltpu.CompilerParams(use_tc_tiling_on_sc=False)


@functools.cache
def _mesh():
    return plsc.VectorSubcoreMesh(core_axis_name="c", subcore_axis_name="s")


def _zero_rows(buf, nrows, ncols):
    def zb(r, carry):
        for c in range(ncols // 16):
            buf[r, pl.ds(c * 16, 16)] = jnp.zeros((16,), jnp.float32)
        return carry

    lax.fori_loop(0, nrows, zb, 0)


def _pipeline(nchunks, srcv, dstv, tab_hbm, acc, bufs, gsems, ssems):
    assert nchunks % NBUF == 0 and nchunks >= 3 * NBUF

    def g_start(j, p):
        pltpu.async_copy(tab_hbm.at[srcv.at[j]], bufs[p], gsems[p])

    def g_wait(j, p):
        pltpu.make_async_copy(tab_hbm.at[srcv.at[j]], bufs[p],
                              gsems[p]).wait()

    def s_start(j, p):
        pltpu.async_copy(bufs[p], acc.at[dstv.at[j]], ssems[p], add=True)

    def s_wait(j, p):
        pltpu.make_async_copy(bufs[p], acc.at[dstv.at[j]], ssems[p]).wait()

    LA = NBUF - 1

    def iter_full(j, p):
        g_wait(j, p)
        s_wait(j - 1, (p - 1) % NBUF)
        s_start(j, p)
        g_start(j + LA, (p + LA) % NBUF)

    for p in range(LA):
        g_start(p, p)
    g_wait(0, 0); s_start(0, 0); g_start(LA, LA)
    for j in range(1, NBUF):
        iter_full(j, j % NBUF)

    def step(g, carry):
        j0 = NBUF * g
        for r in range(NBUF):
            iter_full(j0 + r, r)
        return carry

    lax.fori_loop(1, nchunks // NBUF - 2, step, 0)
    for j in range(nchunks - 2 * NBUF, nchunks):
        p = j % NBUF
        if j + LA < nchunks:
            iter_full(j, p)
        else:
            g_wait(j, p); s_wait(j - 1, (p - 1) % NBUF); s_start(j, p)
    s_wait(nchunks - 1, (nchunks - 1) % NBUF)


def _degrees_call(src_t, dst_t):
    def body(src_hbm, dst_hbm, degi_hbm, dego_hbm, srcv, dstv, onesv, zbuf,
             acc_i, acc_o, *sems):
        semi = sems[:4]
        semo = sems[4:]
        cid = lax.axis_index("c")
        sid = lax.axis_index("s")
        wid = cid * 16 + sid
        pltpu.sync_copy(src_hbm.at[wid], srcv)
        pltpu.sync_copy(dst_hbm.at[wid], dstv)
        for i in range(K // 16):
            onesv[pl.ds(i * 16, 16)] = jnp.full((16,), 1.0, jnp.float32)

        def zb(i, carry):
            zbuf[pl.ds(i * 16, 16)] = jnp.zeros((16,), jnp.float32)
            return carry

        lax.fori_loop(0, RPT // 16, zb, 0)
        pltpu.sync_copy(zbuf, acc_i.at[pl.ds(sid * RPT, RPT)])
        pltpu.sync_copy(zbuf, acc_o.at[pl.ds(sid * RPT, RPT)])
        plsc.subcore_barrier()

        def di_start(j, r):
            pltpu.async_copy(onesv, acc_i.at[dstv.at[j]], semi[r], add=True)

        def di_wait(j, r):
            pltpu.make_async_copy(onesv, acc_i.at[dstv.at[j]],
                                  semi[r]).wait()

        def do_start(j, r):
            pltpu.async_copy(onesv, acc_o.at[srcv.at[j]], semo[r], add=True)

        def do_wait(j, r):
            pltpu.make_async_copy(onesv, acc_o.at[srcv.at[j]],
                                  semo[r]).wait()

        for r in range(4):
            di_start(r, r)
            do_start(r, r)

        def chunk(g, carry):
            j0 = 4 * g
            for r in range(4):
                di_wait(j0 + r - 4, r)
                do_wait(j0 + r - 4, r)
                di_start(j0 + r, r)
                do_start(j0 + r, r)
            return carry

        lax.fori_loop(1, CHUNKS // 4, chunk, 0)
        for r in range(4):
            di_wait(CHUNKS - 4 + r, r)
            do_wait(CHUNKS - 4 + r, r)
        plsc.subcore_barrier()
        sl = pl.ds(sid * RPT, RPT)
        pltpu.sync_copy(acc_i.at[sl], zbuf)
        pltpu.sync_copy(zbuf, degi_hbm.at[cid, sl])
        pltpu.sync_copy(acc_o.at[sl], zbuf)
        pltpu.sync_copy(zbuf, dego_hbm.at[cid, sl])

    f = pl.kernel(
        body,
        out_type=(jax.ShapeDtypeStruct((2, NPAD), jnp.float32),
                  jax.ShapeDtypeStruct((2, NPAD), jnp.float32)),
        mesh=_mesh(),
        scratch_types=[
            pltpu.VMEM((CHUNKS, K), jnp.int32),
            pltpu.VMEM((CHUNKS, K), jnp.int32),
            pltpu.VMEM((K,), jnp.float32),
            pltpu.VMEM((RPT,), jnp.float32),
            pltpu.VMEM_SHARED((NPAD,), jnp.float32),
            pltpu.VMEM_SHARED((NPAD,), jnp.float32),
        ] + [pltpu.SemaphoreType.DMA] * 8,
    )
    return f(src_t, dst_t)


def _msgpass_edge_split(tab, src_t, dst_t, width):
    ex_chunks = [(t * KM, KM) for t in range(RPT // KM)]
    if RPT % KM:
        ex_chunks.append((RPT - RPT % KM, RPT % KM))

    def body(tab_hbm, src_hbm, dst_hbm, out_hbm, srcv, dstv, *rest):
        bufs = rest[:NBUF]
        acc = rest[NBUF]
        gsems = rest[NBUF + 1:2 * NBUF + 1]
        ssems = rest[2 * NBUF + 1:]
        b0 = bufs[0]
        cid = lax.axis_index("c")
        sid = lax.axis_index("s")
        wid = cid * 16 + sid
        pltpu.sync_copy(src_hbm.at[wid], srcv)
        pltpu.sync_copy(dst_hbm.at[wid], dstv)
        _zero_rows(b0, KM, width)
        for t0, n in ex_chunks:
            pltpu.sync_copy(b0.at[pl.ds(0, n)],
                            acc.at[pl.ds(sid * RPT + t0, n)])
        plsc.subcore_barrier()
        _pipeline(CM, srcv, dstv, tab_hbm, acc, bufs, gsems, ssems)
        plsc.subcore_barrier()
        for t0, n in ex_chunks:
            sl = pl.ds(sid * RPT + t0, n)
            pltpu.sync_copy(acc.at[sl], b0.at[pl.ds(0, n)])
            pltpu.sync_copy(b0.at[pl.ds(0, n)], out_hbm.at[cid, sl])

    f = pl.kernel(
        body,
        out_type=jax.ShapeDtypeStruct((2, NPAD, width), jnp.float32),
        mesh=_mesh(),
        scratch_types=(
            [pltpu.VMEM((CM, KM), jnp.int32),
             pltpu.VMEM((CM, KM), jnp.int32)]
            + [pltpu.VMEM((KM, width), jnp.float32)] * NBUF
            + [pltpu.VMEM_SHARED((NPAD, width), jnp.float32)]
            + [pltpu.SemaphoreType.DMA] * (2 * NBUF)
        ),
        compiler_params=_SC_PARAMS,
    )
    return f(tab, src_t, dst_t)


def _mm_scale(x_pad, W1, dego0, dego1):
    def body(x_ref, w_ref, d0, d1, o_ref):
        dinv = lax.rsqrt(d0[...] + d1[...] + 1.0)
        o_ref[...] = jnp.dot(x_ref[...], w_ref[...],
                             preferred_element_type=jnp.float32) * dinv

    return pl.pallas_call(
        body,
        grid=(NPAD // 1024,),
        in_specs=[
            pl.BlockSpec((1024, F_IN), lambda i: (i, 0)),
            pl.BlockSpec((F_IN, H), lambda i: (0, 0)),
            pl.BlockSpec((1024, 1), lambda i: (i, 0)),
            pl.BlockSpec((1024, 1), lambda i: (i, 0)),
        ],
        out_specs=pl.BlockSpec((1024, H), lambda i: (i, 0)),
        out_shape=jax.ShapeDtypeStruct((NPAD, H), jnp.float32),
    )(x_pad, W1, dego0, dego1)


def _mid(p0, p1, degi0, degi1, dego0, dego1, b1, W2p):
    def body(p0_ref, p1_ref, di0, di1, do0, do1, b_ref, w_ref, o_ref):
        dinv_i = lax.rsqrt(di0[...] + di1[...] + 1.0)
        a = (p0_ref[...] + p1_ref[...]) * dinv_i + b_ref[...]
        h = jnp.maximum(a, 0.0)
        dinv_o = lax.rsqrt(do0[...] + do1[...] + 1.0)
        o_ref[...] = jnp.dot(h, w_ref[...],
                             preferred_element_type=jnp.float32) * dinv_o

    return pl.pallas_call(
        body,
        grid=(NPAD // 1024,),
        in_specs=[
            pl.BlockSpec((1024, H), lambda i: (i, 0)),
            pl.BlockSpec((1024, H), lambda i: (i, 0)),
            pl.BlockSpec((1024, 1), lambda i: (i, 0)),
            pl.BlockSpec((1024, 1), lambda i: (i, 0)),
            pl.BlockSpec((1024, 1), lambda i: (i, 0)),
            pl.BlockSpec((1024, 1), lambda i: (i, 0)),
            pl.BlockSpec((1, H), lambda i: (0, 0)),
            pl.BlockSpec((H, CP), lambda i: (0, 0)),
        ],
        out_specs=pl.BlockSpec((1024, CP), lambda i: (i, 0)),
        out_shape=jax.ShapeDtypeStruct((NPAD, CP), jnp.float32),
    )(p0, p1, degi0, degi1, dego0, dego1, b1, W2p)


def _final(q0, q1, degi0, degi1, b2):
    def body(q0_ref, q1_ref, di0, di1, b_ref, o_ref):
        dinv_i = lax.rsqrt(di0[...] + di1[...] + 1.0)
        z = (q0_ref[...] + q1_ref[...])[:, :C] * dinv_i + b_ref[...]
        m = jnp.max(z, axis=1, keepdims=True)
        e = jnp.exp(z - m)
        s = jnp.sum(e, axis=1, keepdims=True)
        o_ref[...] = z - m - jnp.log(s)

    return pl.pallas_call(
        body,
        grid=(10,),
        in_specs=[
            pl.BlockSpec((1000, CP), lambda i: (i, 0)),
            pl.BlockSpec((1000, CP), lambda i: (i, 0)),
            pl.BlockSpec((1000, 1), lambda i: (i, 0)),
            pl.BlockSpec((1000, 1), lambda i: (i, 0)),
            pl.BlockSpec((1, C), lambda i: (0, 0)),
        ],
        out_specs=pl.BlockSpec((1000, C), lambda i: (i, 0)),
        out_shape=jax.ShapeDtypeStruct((N, C), jnp.float32),
    )(q0, q1, degi0, degi1, b2)


def kernel(x, adj, nodes, epoch, W1, b1, W2, b2):
    src = adj[0]
    dst = adj[1]
    ept = E // NTILE

    def shard(idx, chunks, k):
        npad_e = chunks * k - ept
        pad = (jnp.arange(npad_e, dtype=jnp.int32) % (NPAD - N) + N)[None, :]
        pad = jnp.broadcast_to(pad, (NTILE, npad_e))
        t = jnp.concatenate([idx.reshape(NTILE, ept), pad], axis=1)
        return t.reshape(NTILE, chunks, k)

    src_t = shard(src, CHUNKS, K)
    dst_t = shard(dst, CHUNKS, K)
    src_m = shard(src, CM, KM)
    dst_m = shard(dst, CM, KM)

    x_pad = jnp.pad(x, ((0, NPAD - N), (0, 0)))
    W2p = jnp.pad(W2, ((0, 0), (0, CP - C)))

    degi_p, dego_p = _degrees_call(src_t, dst_t)
    degi0 = degi_p[0].reshape(NPAD, 1)
    degi1 = degi_p[1].reshape(NPAD, 1)
    dego0 = dego_p[0].reshape(NPAD, 1)
    dego1 = dego_p[1].reshape(NPAD, 1)

    h1s = _mm_scale(x_pad, W1, dego0, dego1)
    p = _msgpass_edge_split(h1s, src_m, dst_m, H)
    h2s = _mid(p[0], p[1], degi0, degi1, dego0, dego1, b1, W2p)
    q = _msgpass_edge_split(h2s, src_m, dst_m, CP)
    return _final(q[0], q[1], degi0, degi1, b2)


# speedup vs baseline: 28.6932x; 1.0491x over previous
import functools

import jax
import jax.numpy as jnp
from jax import lax
from jax.experimental import pallas as pl
from jax.experimental.pallas import tpu as pltpu
from jax.experimental.pallas import tpu_sc as plsc

N = 10000
E = 320000
F_IN = 128
H = 128
HW = H // 2
C = 40
CP = 48
NPAD = 10240
NTILE = 32
K = 128
CHUNKS = 80
KM1, CM1, NBUF1 = 48, 216, 4
KM2, CM2, NBUF2 = 96, 108, 4
RPT = NPAD // 16

_SC_PARAMS = pltpu.CompilerParams(use_tc_tiling_on_sc=False)


@functools.cache
def _mesh():
    return plsc.VectorSubcoreMesh(core_axis_name="c", subcore_axis_name="s")


def _zero_rows(buf, nrows, ncols):
    def zb(r, carry):
        for c in range(ncols // 16):
            buf[r, pl.ds(c * 16, 16)] = jnp.zeros((16,), jnp.float32)
        return carry

    lax.fori_loop(0, nrows, zb, 0)


def _pipeline(nchunks, srcv, dstv, tab_hbm, acc, bufs, gsems, ssems):
    NBUF = len(bufs)
    assert nchunks % NBUF == 0 and nchunks >= 3 * NBUF

    def g_start(j, p):
        pltpu.async_copy(tab_hbm.at[srcv.at[j]], bufs[p], gsems[p])

    def g_wait(j, p):
        pltpu.make_async_copy(tab_hbm.at[srcv.at[j]], bufs[p],
                              gsems[p]).wait()

    def s_start(j, p):
        pltpu.async_copy(bufs[p], acc.at[dstv.at[j]], ssems[p], add=True)

    def s_wait(j, p):
        pltpu.make_async_copy(bufs[p], acc.at[dstv.at[j]], ssems[p]).wait()

    LA = NBUF - 1

    def iter_full(j, p):
        g_wait(j, p)
        s_wait(j - 1, (p - 1) % NBUF)
        s_start(j, p)
        g_start(j + LA, (p + LA) % NBUF)

    for p in range(LA):
        g_start(p, p)
    g_wait(0, 0); s_start(0, 0); g_start(LA, LA)
    for j in range(1, NBUF):
        iter_full(j, j % NBUF)

    def step(g, carry):
        j0 = NBUF * g
        for r in range(NBUF):
            iter_full(j0 + r, r)
        return carry

    lax.fori_loop(1, nchunks // NBUF - 2, step, 0)
    for j in range(nchunks - 2 * NBUF, nchunks):
        p = j % NBUF
        if j + LA < nchunks:
            iter_full(j, p)
        else:
            g_wait(j, p); s_wait(j - 1, (p - 1) % NBUF); s_start(j, p)
    s_wait(nchunks - 1, (nchunks - 1) % NBUF)


def _degrees_call(src_t, dst_t):
    def body(src_hbm, dst_hbm, degi_hbm, dego_hbm, srcv, dstv, onesv, zbuf,
             acc_i, acc_o, *sems):
        semi = sems[:4]
        semo = sems[4:]
        cid = lax.axis_index("c")
        sid = lax.axis_index("s")
        wid = cid * 16 + sid
        pltpu.sync_copy(src_hbm.at[wid], srcv)
        pltpu.sync_copy(dst_hbm.at[wid], dstv)
        for i in range(K // 16):
            onesv[pl.ds(i * 16, 16)] = jnp.full((16,), 1.0, jnp.float32)

        def zb(i, carry):
            zbuf[pl.ds(i * 16, 16)] = jnp.zeros((16,), jnp.float32)
            return carry

        lax.fori_loop(0, RPT // 16, zb, 0)
        pltpu.sync_copy(zbuf, acc_i.at[pl.ds(sid * RPT, RPT)])
        pltpu.sync_copy(zbuf, acc_o.at[pl.ds(sid * RPT, RPT)])
        plsc.subcore_barrier()

        def di_start(j, r):
            pltpu.async_copy(onesv, acc_i.at[dstv.at[j]], semi[r], add=True)

        def di_wait(j, r):
            pltpu.make_async_copy(onesv, acc_i.at[dstv.at[j]],
                                  semi[r]).wait()

        def do_start(j, r):
            pltpu.async_copy(onesv, acc_o.at[srcv.at[j]], semo[r], add=True)

        def do_wait(j, r):
            pltpu.make_async_copy(onesv, acc_o.at[srcv.at[j]],
                                  semo[r]).wait()

        for r in range(4):
            di_start(r, r)
            do_start(r, r)

        def chunk(g, carry):
            j0 = 4 * g
            for r in range(4):
                di_wait(j0 + r - 4, r)
                do_wait(j0 + r - 4, r)
                di_start(j0 + r, r)
                do_start(j0 + r, r)
            return carry

        lax.fori_loop(1, CHUNKS // 4, chunk, 0)
        for r in range(4):
            di_wait(CHUNKS - 4 + r, r)
            do_wait(CHUNKS - 4 + r, r)
        plsc.subcore_barrier()
        sl = pl.ds(sid * RPT, RPT)
        pltpu.sync_copy(acc_i.at[sl], zbuf)
        pltpu.sync_copy(zbuf, degi_hbm.at[cid, sl])
        pltpu.sync_copy(acc_o.at[sl], zbuf)
        pltpu.sync_copy(zbuf, dego_hbm.at[cid, sl])

    f = pl.kernel(
        body,
        out_type=(jax.ShapeDtypeStruct((2, NPAD), jnp.float32),
                  jax.ShapeDtypeStruct((2, NPAD), jnp.float32)),
        mesh=_mesh(),
        scratch_types=[
            pltpu.VMEM((CHUNKS, K), jnp.int32),
            pltpu.VMEM((CHUNKS, K), jnp.int32),
            pltpu.VMEM((K,), jnp.float32),
            pltpu.VMEM((RPT,), jnp.float32),
            pltpu.VMEM_SHARED((NPAD,), jnp.float32),
            pltpu.VMEM_SHARED((NPAD,), jnp.float32),
        ] + [pltpu.SemaphoreType.DMA] * 8,
    )
    return f(src_t, dst_t)


def _msgpass_edge_split(tab, src_t, dst_t, width, km, cm, nbuf, params):
    ex_chunks = [(t * km, km) for t in range(RPT // km)]
    if RPT % km:
        ex_chunks.append((RPT - RPT % km, RPT % km))

    def body(tab_hbm, src_hbm, dst_hbm, out_hbm, srcv, dstv, *rest):
        bufs = rest[:nbuf]
        acc = rest[nbuf]
        gsems = rest[nbuf + 1:2 * nbuf + 1]
        ssems = rest[2 * nbuf + 1:]
        b0 = bufs[0]
        cid = lax.axis_index("c")
        sid = lax.axis_index("s")
        wid = cid * 16 + sid
        pltpu.sync_copy(src_hbm.at[wid], srcv)
        pltpu.sync_copy(dst_hbm.at[wid], dstv)
        _zero_rows(b0, km, width)
        for t0, n in ex_chunks:
            pltpu.sync_copy(b0.at[pl.ds(0, n)],
                            acc.at[pl.ds(sid * RPT + t0, n)])
        plsc.subcore_barrier()
        _pipeline(cm, srcv, dstv, tab_hbm, acc, bufs, gsems, ssems)
        plsc.subcore_barrier()
        for t0, n in ex_chunks:
            sl = pl.ds(sid * RPT + t0, n)
            pltpu.sync_copy(acc.at[sl], b0.at[pl.ds(0, n)])
            pltpu.sync_copy(b0.at[pl.ds(0, n)], out_hbm.at[cid, sl])

    f = pl.kernel(
        body,
        out_type=jax.ShapeDtypeStruct((2, NPAD, width), jnp.float32),
        mesh=_mesh(),
        scratch_types=(
            [pltpu.VMEM((cm, km), jnp.int32),
             pltpu.VMEM((cm, km), jnp.int32)]
            + [pltpu.VMEM((km, width), jnp.float32)] * nbuf
            + [pltpu.VMEM_SHARED((NPAD, width), jnp.float32)]
            + [pltpu.SemaphoreType.DMA] * (2 * nbuf)
        ),
        compiler_params=params,
    )
    return f(tab, src_t, dst_t)


def _mm_scale(x_pad, W1, dego0, dego1):
    def body(x_ref, w_ref, d0, d1, o_ref):
        dinv = lax.rsqrt(d0[...] + d1[...] + 1.0)
        o_ref[...] = jnp.dot(x_ref[...], w_ref[...],
                             preferred_element_type=jnp.float32) * dinv

    return pl.pallas_call(
        body,
        grid=(NPAD // 1024,),
        in_specs=[
            pl.BlockSpec((1024, F_IN), lambda i: (i, 0)),
            pl.BlockSpec((F_IN, H), lambda i: (0, 0)),
            pl.BlockSpec((1024, 1), lambda i: (i, 0)),
            pl.BlockSpec((1024, 1), lambda i: (i, 0)),
        ],
        out_specs=pl.BlockSpec((1024, H), lambda i: (i, 0)),
        out_shape=jax.ShapeDtypeStruct((NPAD, H), jnp.float32),
    )(x_pad, W1, dego0, dego1)


def _mid(p0, p1, degi0, degi1, dego0, dego1, b1, W2p):
    def body(p0_ref, p1_ref, di0, di1, do0, do1, b_ref, w_ref, o_ref):
        dinv_i = lax.rsqrt(di0[...] + di1[...] + 1.0)
        a = (p0_ref[...] + p1_ref[...]) * dinv_i + b_ref[...]
        h = jnp.maximum(a, 0.0)
        dinv_o = lax.rsqrt(do0[...] + do1[...] + 1.0)
        o_ref[...] = jnp.dot(h, w_ref[...],
                             preferred_element_type=jnp.float32) * dinv_o

    return pl.pallas_call(
        body,
        grid=(NPAD // 1024,),
        in_specs=[
            pl.BlockSpec((1024, H), lambda i: (i, 0)),
            pl.BlockSpec((1024, H), lambda i: (i, 0)),
            pl.BlockSpec((1024, 1), lambda i: (i, 0)),
            pl.BlockSpec((1024, 1), lambda i: (i, 0)),
            pl.BlockSpec((1024, 1), lambda i: (i, 0)),
            pl.BlockSpec((1024, 1), lambda i: (i, 0)),
            pl.BlockSpec((1, H), lambda i: (0, 0)),
            pl.BlockSpec((H, CP), lambda i: (0, 0)),
        ],
        out_specs=pl.BlockSpec((1024, CP), lambda i: (i, 0)),
        out_shape=jax.ShapeDtypeStruct((NPAD, CP), jnp.float32),
    )(p0, p1, degi0, degi1, dego0, dego1, b1, W2p)


def _final(q0, q1, degi0, degi1, b2):
    def body(q0_ref, q1_ref, di0, di1, b_ref, o_ref):
        dinv_i = lax.rsqrt(di0[...] + di1[...] + 1.0)
        z = (q0_ref[...] + q1_ref[...])[:, :C] * dinv_i + b_ref[...]
        m = jnp.max(z, axis=1, keepdims=True)
        e = jnp.exp(z - m)
        s = jnp.sum(e, axis=1, keepdims=True)
        o_ref[...] = z - m - jnp.log(s)

    return pl.pallas_call(
        body,
        grid=(10,),
        in_specs=[
            pl.BlockSpec((1000, CP), lambda i: (i, 0)),
            pl.BlockSpec((1000, CP), lambda i: (i, 0)),
            pl.BlockSpec((1000, 1), lambda i: (i, 0)),
            pl.BlockSpec((1000, 1), lambda i: (i, 0)),
            pl.BlockSpec((1, C), lambda i: (0, 0)),
        ],
        out_specs=pl.BlockSpec((1000, C), lambda i: (i, 0)),
        out_shape=jax.ShapeDtypeStruct((N, C), jnp.float32),
    )(q0, q1, degi0, degi1, b2)


def kernel(x, adj, nodes, epoch, W1, b1, W2, b2):
    src = adj[0]
    dst = adj[1]
    ept = E // NTILE

    def shard(idx, chunks, k):
        npad_e = chunks * k - ept
        pad = (jnp.arange(npad_e, dtype=jnp.int32) % (NPAD - N) + N)[None, :]
        pad = jnp.broadcast_to(pad, (NTILE, npad_e))
        t = jnp.concatenate([idx.reshape(NTILE, ept), pad], axis=1)
        return t.reshape(NTILE, chunks, k)

    src_t = shard(src, CHUNKS, K)
    dst_t = shard(dst, CHUNKS, K)
    src_m1 = shard(src, CM1, KM1)
    dst_m1 = shard(dst, CM1, KM1)
    src_m2 = shard(src, CM2, KM2)
    dst_m2 = shard(dst, CM2, KM2)

    x_pad = jnp.pad(x, ((0, NPAD - N), (0, 0)))
    W2p = jnp.pad(W2, ((0, 0), (0, CP - C)))

    degi_p, dego_p = _degrees_call(src_t, dst_t)
    degi0 = degi_p[0].reshape(NPAD, 1)
    degi1 = degi_p[1].reshape(NPAD, 1)
    dego0 = dego_p[0].reshape(NPAD, 1)
    dego1 = dego_p[1].reshape(NPAD, 1)

    h1s = _mm_scale(x_pad, W1, dego0, dego1)
    p = _msgpass_edge_split(h1s, src_m1, dst_m1, H, KM1, CM1, NBUF1,
                            _SC_PARAMS)
    h2s = _mid(p[0], p[1], degi0, degi1, dego0, dego1, b1, W2p)
    q = _msgpass_edge_split(h2s, src_m2, dst_m2, CP, KM2, CM2, NBUF2,
                            _SC_PARAMS)
    return _final(q[0], q[1], degi0, degi1, b2)
